# baseline (speedup 1.0000x reference)
import functools

import jax
import jax.numpy as jnp
from jax import lax
from jax.experimental import pallas as pl
from jax.experimental.pallas import tpu as pltpu
from jax.experimental.pallas import tpu_sc as plsc

_D = 64
_NC = 2
_NS = 16
_NW = _NC * _NS
_CHUNK = 128


def _mm2_body(x_ref, wl_ref, wr_ref, y_ref, r_ref):
    xb = x_ref[...]
    y_ref[...] = jnp.dot(xb, wl_ref[...], preferred_element_type=jnp.float32)
    r_ref[...] = jnp.dot(xb, wr_ref[...], preferred_element_type=jnp.float32)


def _mm2(x, wl, wr, blk=1024):
    n, d_in = x.shape
    return pl.pallas_call(
        _mm2_body,
        grid=(n // blk,),
        in_specs=[
            pl.BlockSpec((blk, d_in), lambda i: (i, 0)),
            pl.BlockSpec((d_in, _D), lambda i: (0, 0)),
            pl.BlockSpec((d_in, _D), lambda i: (0, 0)),
        ],
        out_specs=[
            pl.BlockSpec((blk, _D), lambda i: (i, 0)),
            pl.BlockSpec((blk, _D), lambda i: (i, 0)),
        ],
        out_shape=[jax.ShapeDtypeStruct((n, _D), jnp.float32)] * 2,
    )(x, wl, wr)


def _combine_body(pa_ref, pb_ref, ca_ref, cb_ref, r_ref, wl_ref, wr_ref,
                  b_ref, y_ref, r2_ref):
    cnt = jnp.maximum(ca_ref[...] + cb_ref[...], 1.0)
    h = (pa_ref[...] + pb_ref[...]) / cnt + r_ref[...] + b_ref[...]
    h = jnp.maximum(h, 0.0)
    y_ref[...] = jnp.dot(h, wl_ref[...], preferred_element_type=jnp.float32)
    r2_ref[...] = jnp.dot(h, wr_ref[...], preferred_element_type=jnp.float32)


def _combine(pa, pb, ca, cb, r, wl, wr, b, blk=1024):
    n = pa.shape[0]
    return pl.pallas_call(
        _combine_body,
        grid=(n // blk,),
        in_specs=[
            pl.BlockSpec((blk, _D), lambda i: (i, 0)),
            pl.BlockSpec((blk, _D), lambda i: (i, 0)),
            pl.BlockSpec((blk, 1), lambda i: (i, 0)),
            pl.BlockSpec((blk, 1), lambda i: (i, 0)),
            pl.BlockSpec((blk, _D), lambda i: (i, 0)),
            pl.BlockSpec((_D, _D), lambda i: (0, 0)),
            pl.BlockSpec((_D, _D), lambda i: (0, 0)),
            pl.BlockSpec((1, _D), lambda i: (0, 0)),
        ],
        out_specs=[
            pl.BlockSpec((blk, _D), lambda i: (i, 0)),
            pl.BlockSpec((blk, _D), lambda i: (i, 0)),
        ],
        out_shape=[jax.ShapeDtypeStruct((n, _D), jnp.float32)] * 2,
    )(pa, pb, ca, cb, r, wl, wr, b)


def _final_body(pa_ref, pb_ref, ca_ref, cb_ref, r_ref, b_ref, out_ref):
    cnt = jnp.maximum(ca_ref[...] + cb_ref[...], 1.0)
    out_ref[...] = (pa_ref[...] + pb_ref[...]) / cnt + r_ref[...] + b_ref[...]


def _final(pa, pb, ca, cb, r, b, blk=1024):
    n = pa.shape[0]
    return pl.pallas_call(
        _final_body,
        grid=(n // blk,),
        in_specs=[
            pl.BlockSpec((blk, _D), lambda i: (i, 0)),
            pl.BlockSpec((blk, _D), lambda i: (i, 0)),
            pl.BlockSpec((blk, 1), lambda i: (i, 0)),
            pl.BlockSpec((blk, 1), lambda i: (i, 0)),
            pl.BlockSpec((blk, _D), lambda i: (i, 0)),
            pl.BlockSpec((1, _D), lambda i: (0, 0)),
        ],
        out_specs=pl.BlockSpec((blk, _D), lambda i: (i, 0)),
        out_shape=jax.ShapeDtypeStruct((n, _D), jnp.float32),
    )(pa, pb, ca, cb, r, b)


def _sc_agg_build(n_pad, k, with_counts):
    rpt = n_pad // _NS

    def body(*refs):
        if with_counts:
            (y_hbm, src_hbm, dst_hbm, zrow_hbm, zcnt_hbm, ones_hbm,
             part_hbm, cnt_hbm,
             acc_sh, cnt_sh, y_sh, srcv, dstv, rows0, rows1, rows2,
             onesv, gsem, ssem, csem) = refs
        else:
            (y_hbm, src_hbm, dst_hbm, zrow_hbm,
             part_hbm,
             acc_sh, y_sh, srcv, dstv, rows0, rows1, rows2,
             gsem, ssem, csem) = refs
        cid = lax.axis_index("c")
        sid = lax.axis_index("s")
        wid = cid * _NS + sid
        base = sid * rpt

        pro = [
            pltpu.make_async_copy(zrow_hbm, acc_sh.at[pl.ds(base, rpt)],
                                  ssem),
            pltpu.make_async_copy(y_hbm.at[pl.ds(base, rpt)],
                                  y_sh.at[pl.ds(base, rpt)], ssem),
            pltpu.make_async_copy(src_hbm.at[wid], srcv, ssem),
            pltpu.make_async_copy(dst_hbm.at[wid], dstv, ssem),
        ]
        if with_counts:
            pro.append(pltpu.make_async_copy(
                zcnt_hbm, cnt_sh.at[pl.ds(base, rpt)], ssem))
            pro.append(pltpu.make_async_copy(ones_hbm, onesv, ssem))
        for d in pro:
            d.start()
        for d in pro:
            d.wait()
        plsc.subcore_barrier()

        bufs = (rows0, rows1, rows2)
        ng3 = k // 3

        def gstart(j, buf):
            pltpu.async_copy(y_sh.at[srcv.at[j]], buf, gsem)

        def gwait(buf):
            pltpu.make_async_copy(y_sh.at[srcv.at[0]], buf, gsem).wait()

        def sstart(j, buf):
            if with_counts:
                pltpu.make_async_copy(
                    onesv, cnt_sh.at[dstv.at[j]], csem).start(add=True)
            pltpu.make_async_copy(
                buf, acc_sh.at[dstv.at[j]], ssem).start(add=True)

        def swait():
            pltpu.make_async_copy(
                rows0, acc_sh.at[dstv.at[0]], ssem).wait()

        gstart(0, bufs[0])
        gstart(1, bufs[1])
        gwait(bufs[0])
        sstart(0, bufs[0])
        gstart(2, bufs[2])
        for j in (1, 2):
            gwait(bufs[j % 3])
            sstart(j, bufs[j % 3])
            swait()
            gstart(j + 2, bufs[(j + 2) % 3])

        def group(g, carry):
            for b in range(3):
                j3 = 3 * g + b
                gwait(bufs[b])
                sstart(j3, bufs[b])
                swait()
                gstart(j3 + 2, bufs[(b + 2) % 3])
            return carry

        lax.fori_loop(1, ng3 - 1, group, 0)

        jt = k - 3
        gwait(bufs[jt % 3])
        sstart(jt, bufs[jt % 3])
        swait()
        gstart(jt + 2, bufs[(jt + 2) % 3])
        for j in (k - 2, k - 1):
            gwait(bufs[j % 3])
            sstart(j, bufs[j % 3])
            swait()
        swait()
        if with_counts:
            def cdrain(j, carry):
                pltpu.make_async_copy(
                    onesv, cnt_sh.at[dstv.at[0]], csem).wait()
                return carry
            lax.fori_loop(0, k, cdrain, 0)
        plsc.subcore_barrier()

        out_base = cid * n_pad + base
        pltpu.sync_copy(acc_sh.at[pl.ds(base, rpt)],
                        part_hbm.at[pl.ds(out_base, rpt)])
        if with_counts:
            pltpu.sync_copy(cnt_sh.at[pl.ds(base, rpt)],
                            cnt_hbm.at[pl.ds(out_base, rpt)])

    mesh = plsc.VectorSubcoreMesh(core_axis_name="c", subcore_axis_name="s",
                                  num_cores=_NC, num_subcores=_NS)
    out_type = [jax.ShapeDtypeStruct((_NC * n_pad, _D), jnp.float32)]
    scratch = [
        pltpu.VMEM_SHARED((n_pad, _D), jnp.float32),
        pltpu.VMEM_SHARED((n_pad, _D), jnp.float32),
        pltpu.VMEM((k, _CHUNK), jnp.int32),
        pltpu.VMEM((k, _CHUNK), jnp.int32),
        pltpu.VMEM((_CHUNK, _D), jnp.float32),
        pltpu.VMEM((_CHUNK, _D), jnp.float32),
        pltpu.VMEM((_CHUNK, _D), jnp.float32),
        pltpu.SemaphoreType.DMA,
        pltpu.SemaphoreType.DMA,
        pltpu.SemaphoreType.DMA,
    ]
    if with_counts:
        out_type.append(jax.ShapeDtypeStruct((_NC * n_pad,), jnp.float32))
        scratch = [
            pltpu.VMEM_SHARED((n_pad, _D), jnp.float32),
            pltpu.VMEM_SHARED((n_pad,), jnp.float32),
            pltpu.VMEM_SHARED((n_pad, _D), jnp.float32),
            pltpu.VMEM((k, _CHUNK), jnp.int32),
            pltpu.VMEM((k, _CHUNK), jnp.int32),
            pltpu.VMEM((_CHUNK, _D), jnp.float32),
            pltpu.VMEM((_CHUNK, _D), jnp.float32),
            pltpu.VMEM((_CHUNK, _D), jnp.float32),
            pltpu.VMEM((_CHUNK,), jnp.float32),
            pltpu.SemaphoreType.DMA,
            pltpu.SemaphoreType.DMA,
            pltpu.SemaphoreType.DMA,
        ]
    return pl.kernel(body, out_type=out_type, mesh=mesh,
                     scratch_types=scratch,
                     compiler_params=pltpu.CompilerParams(
                         use_tc_tiling_on_sc=False))


def kernel(x, edge_index, Wl1, Wr1, b1, Wl2, Wr2, b2):
    n, d_in = x.shape
    e = edge_index.shape[1]
    n_pad = ((n + 1 + _NS * _D - 1) // (_NS * _D)) * (_NS * _D)
    k = -(-e // (_NW * _CHUNK))
    k = max(-(-k // 3) * 3, 9)
    e_pad = _NW * k * _CHUNK

    src = edge_index[0]
    dst = edge_index[1]
    dummy = n + (jnp.arange(e_pad - e, dtype=jnp.int32) % (n_pad - n))
    src_p = jnp.concatenate(
        [src, jnp.zeros((e_pad - e,), jnp.int32)]).reshape(_NW, k, _CHUNK)
    dst_p = jnp.concatenate([dst, dummy]).reshape(_NW, k, _CHUNK)

    x_p = jnp.pad(x, ((0, n_pad - n), (0, 0)))
    zrow = jnp.zeros((n_pad // _NS, _D), jnp.float32)
    zcnt = jnp.zeros((n_pad // _NS,), jnp.float32)
    ones = jnp.ones((_CHUNK,), jnp.float32)
    b1r = b1.reshape(1, _D)
    b2r = b2.reshape(1, _D)

    y1, r1 = _mm2(x_p, Wl1, Wr1)
    agg1 = _sc_agg_build(n_pad, k, True)
    part1, cnt1 = agg1(y1, src_p, dst_p, zrow, zcnt, ones)
    pa1, pb1 = part1[:n_pad], part1[n_pad:]
    ca = cnt1[:n_pad].reshape(n_pad, 1)
    cb = cnt1[n_pad:].reshape(n_pad, 1)

    y2, r2 = _combine(pa1, pb1, ca, cb, r1, Wl2, Wr2, b1r)

    agg2 = _sc_agg_build(n_pad, k, False)
    (part2,) = agg2(y2, src_p, dst_p, zrow)
    out = _final(part2[:n_pad], part2[n_pad:], ca, cb, r2, b2r)
    return out[:n]


# speedup vs baseline: 1.0172x; 1.0172x over previous
import functools

import jax
import jax.numpy as jnp
from jax import lax
from jax.experimental import pallas as pl
from jax.experimental.pallas import tpu as pltpu
from jax.experimental.pallas import tpu_sc as plsc

_D = 64
_NC = 2
_NS = 16
_NW = _NC * _NS
_CHUNK = 96


def _mm1_body(x_ref, w_ref, o_ref):
    o_ref[...] = jnp.dot(x_ref[...], w_ref[...],
                         preferred_element_type=jnp.float32)


def _mm1(x, w, blk=1024):
    n, d_in = x.shape
    return pl.pallas_call(
        _mm1_body,
        grid=(n // blk,),
        in_specs=[
            pl.BlockSpec((blk, d_in), lambda i: (i, 0)),
            pl.BlockSpec((d_in, _D), lambda i: (0, 0)),
        ],
        out_specs=pl.BlockSpec((blk, _D), lambda i: (i, 0)),
        out_shape=jax.ShapeDtypeStruct((n, _D), jnp.float32),
    )(x, w)


def _combine_body(pa_ref, pb_ref, ca_ref, cb_ref, r_ref, b_ref, h_ref):
    cnt = jnp.maximum(ca_ref[...] + cb_ref[...], 1.0)
    h = (pa_ref[...] + pb_ref[...]) / cnt + r_ref[...] + b_ref[...]
    h_ref[...] = jnp.maximum(h, 0.0)


def _combine(pa, pb, ca, cb, r, b, blk=1024):
    n = pa.shape[0]
    return pl.pallas_call(
        _combine_body,
        grid=(n // blk,),
        in_specs=[
            pl.BlockSpec((blk, _D), lambda i: (i, 0)),
            pl.BlockSpec((blk, _D), lambda i: (i, 0)),
            pl.BlockSpec((blk, 1), lambda i: (i, 0)),
            pl.BlockSpec((blk, 1), lambda i: (i, 0)),
            pl.BlockSpec((blk, _D), lambda i: (i, 0)),
            pl.BlockSpec((1, _D), lambda i: (0, 0)),
        ],
        out_specs=pl.BlockSpec((blk, _D), lambda i: (i, 0)),
        out_shape=jax.ShapeDtypeStruct((n, _D), jnp.float32),
    )(pa, pb, ca, cb, r, b)


def _final_body(pa_ref, pb_ref, ca_ref, cb_ref, r_ref, b_ref, out_ref):
    cnt = jnp.maximum(ca_ref[...] + cb_ref[...], 1.0)
    out_ref[...] = (pa_ref[...] + pb_ref[...]) / cnt + r_ref[...] + b_ref[...]


def _final(pa, pb, ca, cb, r, b, blk=1024):
    n = pa.shape[0]
    return pl.pallas_call(
        _final_body,
        grid=(n // blk,),
        in_specs=[
            pl.BlockSpec((blk, _D), lambda i: (i, 0)),
            pl.BlockSpec((blk, _D), lambda i: (i, 0)),
            pl.BlockSpec((blk, 1), lambda i: (i, 0)),
            pl.BlockSpec((blk, 1), lambda i: (i, 0)),
            pl.BlockSpec((blk, _D), lambda i: (i, 0)),
            pl.BlockSpec((1, _D), lambda i: (0, 0)),
        ],
        out_specs=pl.BlockSpec((blk, _D), lambda i: (i, 0)),
        out_shape=jax.ShapeDtypeStruct((n, _D), jnp.float32),
    )(pa, pb, ca, cb, r, b)


def _sc_agg_build(n_pad, k, with_counts):
    rpt = n_pad // _NS

    def body(*refs):
        if with_counts:
            (y_hbm, src_hbm, dst_hbm, zrow_hbm, zcnt_hbm, ones_hbm,
             part_hbm, cnt_hbm,
             acc_sh, cnt_sh, y_sh, srcv, dstv, rows0, rows1, rows2,
             onesv, gsem, ssem, csem) = refs
        else:
            (y_hbm, src_hbm, dst_hbm, zrow_hbm,
             part_hbm,
             acc_sh, y_sh, srcv, dstv, rows0, rows1, rows2,
             gsem, ssem, csem) = refs
        cid = lax.axis_index("c")
        sid = lax.axis_index("s")
        wid = cid * _NS + sid
        base = sid * rpt

        pro = [
            pltpu.make_async_copy(zrow_hbm, acc_sh.at[pl.ds(base, rpt)],
                                  ssem),
            pltpu.make_async_copy(y_hbm.at[pl.ds(base, rpt)],
                                  y_sh.at[pl.ds(base, rpt)], ssem),
            pltpu.make_async_copy(src_hbm.at[wid], srcv, ssem),
            pltpu.make_async_copy(dst_hbm.at[wid], dstv, ssem),
        ]
        if with_counts:
            pro.append(pltpu.make_async_copy(
                zcnt_hbm, cnt_sh.at[pl.ds(base, rpt)], ssem))
            pro.append(pltpu.make_async_copy(ones_hbm, onesv, ssem))
        for d in pro:
            d.start()
        for d in pro:
            d.wait()
        plsc.subcore_barrier()

        bufs = (rows0, rows1, rows2)
        ng3 = k // 3

        def gstart(j, buf):
            pltpu.async_copy(y_sh.at[srcv.at[j]], buf, gsem)

        def gwait(buf):
            pltpu.make_async_copy(y_sh.at[srcv.at[0]], buf, gsem).wait()

        def sstart(j, buf):
            if with_counts:
                pltpu.make_async_copy(
                    onesv, cnt_sh.at[dstv.at[j]], csem).start(add=True)
            pltpu.make_async_copy(
                buf, acc_sh.at[dstv.at[j]], ssem).start(add=True)

        def swait():
            pltpu.make_async_copy(
                rows0, acc_sh.at[dstv.at[0]], ssem).wait()

        gstart(0, bufs[0])
        gstart(1, bufs[1])
        gwait(bufs[0])
        sstart(0, bufs[0])
        gstart(2, bufs[2])
        for j in (1, 2):
            gwait(bufs[j % 3])
            sstart(j, bufs[j % 3])
            swait()
            gstart(j + 2, bufs[(j + 2) % 3])

        def group(g, carry):
            for b in range(3):
                j3 = 3 * g + b
                gwait(bufs[b])
                sstart(j3, bufs[b])
                swait()
                gstart(j3 + 2, bufs[(b + 2) % 3])
            return carry

        lax.fori_loop(1, ng3 - 1, group, 0)

        jt = k - 3
        gwait(bufs[jt % 3])
        sstart(jt, bufs[jt % 3])
        swait()
        gstart(jt + 2, bufs[(jt + 2) % 3])
        for j in (k - 2, k - 1):
            gwait(bufs[j % 3])
            sstart(j, bufs[j % 3])
            swait()
        swait()
        if with_counts:
            def cdrain(j, carry):
                pltpu.make_async_copy(
                    onesv, cnt_sh.at[dstv.at[0]], csem).wait()
                return carry
            lax.fori_loop(0, k, cdrain, 0)
        plsc.subcore_barrier()

        out_base = cid * n_pad + base
        pltpu.sync_copy(acc_sh.at[pl.ds(base, rpt)],
                        part_hbm.at[pl.ds(out_base, rpt)])
        if with_counts:
            pltpu.sync_copy(cnt_sh.at[pl.ds(base, rpt)],
                            cnt_hbm.at[pl.ds(out_base, rpt)])

    mesh = plsc.VectorSubcoreMesh(core_axis_name="c", subcore_axis_name="s",
                                  num_cores=_NC, num_subcores=_NS)
    out_type = [jax.ShapeDtypeStruct((_NC * n_pad, _D), jnp.float32)]
    scratch = [
        pltpu.VMEM_SHARED((n_pad, _D), jnp.float32),
        pltpu.VMEM_SHARED((n_pad, _D), jnp.float32),
        pltpu.VMEM((k, _CHUNK), jnp.int32),
        pltpu.VMEM((k, _CHUNK), jnp.int32),
        pltpu.VMEM((_CHUNK, _D), jnp.float32),
        pltpu.VMEM((_CHUNK, _D), jnp.float32),
        pltpu.VMEM((_CHUNK, _D), jnp.float32),
        pltpu.SemaphoreType.DMA,
        pltpu.SemaphoreType.DMA,
        pltpu.SemaphoreType.DMA,
    ]
    if with_counts:
        out_type.append(jax.ShapeDtypeStruct((_NC * n_pad,), jnp.float32))
        scratch = [
            pltpu.VMEM_SHARED((n_pad, _D), jnp.float32),
            pltpu.VMEM_SHARED((n_pad,), jnp.float32),
            pltpu.VMEM_SHARED((n_pad, _D), jnp.float32),
            pltpu.VMEM((k, _CHUNK), jnp.int32),
            pltpu.VMEM((k, _CHUNK), jnp.int32),
            pltpu.VMEM((_CHUNK, _D), jnp.float32),
            pltpu.VMEM((_CHUNK, _D), jnp.float32),
            pltpu.VMEM((_CHUNK, _D), jnp.float32),
            pltpu.VMEM((_CHUNK,), jnp.float32),
            pltpu.SemaphoreType.DMA,
            pltpu.SemaphoreType.DMA,
            pltpu.SemaphoreType.DMA,
        ]
    return pl.kernel(body, out_type=out_type, mesh=mesh,
                     scratch_types=scratch,
                     compiler_params=pltpu.CompilerParams(
                         use_tc_tiling_on_sc=False))


def kernel(x, edge_index, Wl1, Wr1, b1, Wl2, Wr2, b2):
    n, d_in = x.shape
    e = edge_index.shape[1]
    n_pad = ((n + 1 + _NS * _D - 1) // (_NS * _D)) * (_NS * _D)
    k = -(-e // (_NW * _CHUNK))
    k = max(-(-k // 3) * 3, 9)
    e_pad = _NW * k * _CHUNK

    src = edge_index[0]
    dst = edge_index[1]
    dummy = n + (jnp.arange(e_pad - e, dtype=jnp.int32) % (n_pad - n))
    src_p = jnp.concatenate(
        [src, jnp.zeros((e_pad - e,), jnp.int32)]).reshape(_NW, k, _CHUNK)
    dst_p = jnp.concatenate([dst, dummy]).reshape(_NW, k, _CHUNK)

    x_p = jnp.pad(x, ((0, n_pad - n), (0, 0)))
    zrow = jnp.zeros((n_pad // _NS, _D), jnp.float32)
    zcnt = jnp.zeros((n_pad // _NS,), jnp.float32)
    ones = jnp.ones((_CHUNK,), jnp.float32)
    b1r = b1.reshape(1, _D)
    b2r = b2.reshape(1, _D)

    y1 = _mm1(x_p, Wl1)
    agg1 = _sc_agg_build(n_pad, k, True)
    part1, cnt1 = agg1(y1, src_p, dst_p, zrow, zcnt, ones)
    r1 = _mm1(x_p, Wr1)
    pa1, pb1 = part1[:n_pad], part1[n_pad:]
    ca = cnt1[:n_pad].reshape(n_pad, 1)
    cb = cnt1[n_pad:].reshape(n_pad, 1)

    h = _combine(pa1, pb1, ca, cb, r1, b1r)
    y2 = _mm1(h, Wl2)

    agg2 = _sc_agg_build(n_pad, k, False)
    (part2,) = agg2(y2, src_p, dst_p, zrow)
    r2 = _mm1(h, Wr2)
    out = _final(part2[:n_pad], part2[n_pad:], ca, cb, r2, b2r)
    return out[:n]


# speedup vs baseline: 1.0601x; 1.0422x over previous
import functools

import jax
import jax.numpy as jnp
from jax import lax
from jax.experimental import pallas as pl
from jax.experimental.pallas import tpu as pltpu
from jax.experimental.pallas import tpu_sc as plsc

_D = 64
_NC = 2
_NS = 16
_NW = _NC * _NS
_CHUNK = 96


def _mm2_body(x_ref, wl_ref, wr_ref, y_ref, r_ref):
    xb = x_ref[...]
    y_ref[...] = jnp.dot(xb, wl_ref[...], preferred_element_type=jnp.float32)
    r_ref[...] = jnp.dot(xb, wr_ref[...], preferred_element_type=jnp.float32)


def _mm2(x, wl, wr, blk=1024):
    n, d_in = x.shape
    return pl.pallas_call(
        _mm2_body,
        grid=(n // blk,),
        in_specs=[
            pl.BlockSpec((blk, d_in), lambda i: (i, 0)),
            pl.BlockSpec((d_in, _D), lambda i: (0, 0)),
            pl.BlockSpec((d_in, _D), lambda i: (0, 0)),
        ],
        out_specs=[
            pl.BlockSpec((blk, _D), lambda i: (i, 0)),
            pl.BlockSpec((blk, _D), lambda i: (i, 0)),
        ],
        out_shape=[jax.ShapeDtypeStruct((n, _D), jnp.float32)] * 2,
    )(x, wl, wr)


def _combine_body(pa_ref, pb_ref, ca_ref, cb_ref, r_ref, wl_ref, wr_ref,
                  b_ref, y_ref, r2_ref):
    cnt = jnp.maximum(ca_ref[...] + cb_ref[...], 1.0)
    h = (pa_ref[...] + pb_ref[...]) / cnt + r_ref[...] + b_ref[...]
    h = jnp.maximum(h, 0.0)
    y_ref[...] = jnp.dot(h, wl_ref[...], preferred_element_type=jnp.float32)
    r2_ref[...] = jnp.dot(h, wr_ref[...], preferred_element_type=jnp.float32)


def _combine(pa, pb, ca, cb, r, wl, wr, b, blk=1024):
    n = pa.shape[0]
    return pl.pallas_call(
        _combine_body,
        grid=(n // blk,),
        in_specs=[
            pl.BlockSpec((blk, _D), lambda i: (i, 0)),
            pl.BlockSpec((blk, _D), lambda i: (i, 0)),
            pl.BlockSpec((blk, 1), lambda i: (i, 0)),
            pl.BlockSpec((blk, 1), lambda i: (i, 0)),
            pl.BlockSpec((blk, _D), lambda i: (i, 0)),
            pl.BlockSpec((_D, _D), lambda i: (0, 0)),
            pl.BlockSpec((_D, _D), lambda i: (0, 0)),
            pl.BlockSpec((1, _D), lambda i: (0, 0)),
        ],
        out_specs=[
            pl.BlockSpec((blk, _D), lambda i: (i, 0)),
            pl.BlockSpec((blk, _D), lambda i: (i, 0)),
        ],
        out_shape=[jax.ShapeDtypeStruct((n, _D), jnp.float32)] * 2,
    )(pa, pb, ca, cb, r, wl, wr, b)


def _final_body(pa_ref, pb_ref, ca_ref, cb_ref, r_ref, b_ref, out_ref):
    cnt = jnp.maximum(ca_ref[...] + cb_ref[...], 1.0)
    out_ref[...] = (pa_ref[...] + pb_ref[...]) / cnt + r_ref[...] + b_ref[...]


def _final(pa, pb, ca, cb, r, b, blk=1024):
    n = pa.shape[0]
    return pl.pallas_call(
        _final_body,
        grid=(n // blk,),
        in_specs=[
            pl.BlockSpec((blk, _D), lambda i: (i, 0)),
            pl.BlockSpec((blk, _D), lambda i: (i, 0)),
            pl.BlockSpec((blk, 1), lambda i: (i, 0)),
            pl.BlockSpec((blk, 1), lambda i: (i, 0)),
            pl.BlockSpec((blk, _D), lambda i: (i, 0)),
            pl.BlockSpec((1, _D), lambda i: (0, 0)),
        ],
        out_specs=pl.BlockSpec((blk, _D), lambda i: (i, 0)),
        out_shape=jax.ShapeDtypeStruct((n, _D), jnp.float32),
    )(pa, pb, ca, cb, r, b)


def _sc_agg_build(n_pad, k, with_counts):
    rpt = n_pad // _NS

    def body(*refs):
        if with_counts:
            (y_hbm, src_hbm, dst_hbm, zrow_hbm, zcnt_hbm, ones_hbm,
             part_hbm, cnt_hbm,
             acc_sh, cnt_sh, y_sh, srcv, dstv, rows0, rows1, rows2,
             onesv, gsem, ssem, csem) = refs
        else:
            (y_hbm, src_hbm, dst_hbm, zrow_hbm,
             part_hbm,
             acc_sh, y_sh, srcv, dstv, rows0, rows1, rows2,
             gsem, ssem, csem) = refs
        cid = lax.axis_index("c")
        sid = lax.axis_index("s")
        wid = cid * _NS + sid
        base = sid * rpt

        pro = [
            pltpu.make_async_copy(zrow_hbm, acc_sh.at[pl.ds(base, rpt)],
                                  ssem),
            pltpu.make_async_copy(y_hbm.at[pl.ds(base, rpt)],
                                  y_sh.at[pl.ds(base, rpt)], ssem),
            pltpu.make_async_copy(src_hbm.at[wid], srcv, ssem),
            pltpu.make_async_copy(dst_hbm.at[wid], dstv, ssem),
        ]
        if with_counts:
            pro.append(pltpu.make_async_copy(
                zcnt_hbm, cnt_sh.at[pl.ds(base, rpt)], ssem))
            pro.append(pltpu.make_async_copy(ones_hbm, onesv, ssem))
        for d in pro:
            d.start()
        for d in pro:
            d.wait()
        plsc.subcore_barrier()

        bufs = (rows0, rows1, rows2)
        ng3 = k // 3

        def gstart(j, buf):
            pltpu.async_copy(y_sh.at[srcv.at[j]], buf, gsem)

        def gwait(buf):
            pltpu.make_async_copy(y_sh.at[srcv.at[0]], buf, gsem).wait()

        def sstart(j, buf):
            if with_counts:
                pltpu.make_async_copy(
                    onesv, cnt_sh.at[dstv.at[j]], csem).start(add=True)
            pltpu.make_async_copy(
                buf, acc_sh.at[dstv.at[j]], ssem).start(add=True)

        def swait():
            pltpu.make_async_copy(
                rows0, acc_sh.at[dstv.at[0]], ssem).wait()

        gstart(0, bufs[0])
        gstart(1, bufs[1])
        gwait(bufs[0])
        sstart(0, bufs[0])
        gstart(2, bufs[2])
        for j in (1, 2):
            gwait(bufs[j % 3])
            sstart(j, bufs[j % 3])
            swait()
            gstart(j + 2, bufs[(j + 2) % 3])

        def group(g, carry):
            for b in range(3):
                j3 = 3 * g + b
                gwait(bufs[b])
                sstart(j3, bufs[b])
                swait()
                gstart(j3 + 2, bufs[(b + 2) % 3])
            return carry

        lax.fori_loop(1, ng3 - 1, group, 0)

        jt = k - 3
        gwait(bufs[jt % 3])
        sstart(jt, bufs[jt % 3])
        swait()
        gstart(jt + 2, bufs[(jt + 2) % 3])
        for j in (k - 2, k - 1):
            gwait(bufs[j % 3])
            sstart(j, bufs[j % 3])
            swait()
        swait()
        if with_counts:
            def cdrain(j, carry):
                pltpu.make_async_copy(
                    onesv, cnt_sh.at[dstv.at[0]], csem).wait()
                return carry
            lax.fori_loop(0, k, cdrain, 0)
        plsc.subcore_barrier()

        out_base = cid * n_pad + base
        pltpu.sync_copy(acc_sh.at[pl.ds(base, rpt)],
                        part_hbm.at[pl.ds(out_base, rpt)])
        if with_counts:
            pltpu.sync_copy(cnt_sh.at[pl.ds(base, rpt)],
                            cnt_hbm.at[pl.ds(out_base, rpt)])

    mesh = plsc.VectorSubcoreMesh(core_axis_name="c", subcore_axis_name="s",
                                  num_cores=_NC, num_subcores=_NS)
    out_type = [jax.ShapeDtypeStruct((_NC * n_pad, _D), jnp.float32)]
    scratch = [
        pltpu.VMEM_SHARED((n_pad, _D), jnp.float32),
        pltpu.VMEM_SHARED((n_pad, _D), jnp.float32),
        pltpu.VMEM((k, _CHUNK), jnp.int32),
        pltpu.VMEM((k, _CHUNK), jnp.int32),
        pltpu.VMEM((_CHUNK, _D), jnp.float32),
        pltpu.VMEM((_CHUNK, _D), jnp.float32),
        pltpu.VMEM((_CHUNK, _D), jnp.float32),
        pltpu.SemaphoreType.DMA,
        pltpu.SemaphoreType.DMA,
        pltpu.SemaphoreType.DMA,
    ]
    if with_counts:
        out_type.append(jax.ShapeDtypeStruct((_NC * n_pad,), jnp.float32))
        scratch = [
            pltpu.VMEM_SHARED((n_pad, _D), jnp.float32),
            pltpu.VMEM_SHARED((n_pad,), jnp.float32),
            pltpu.VMEM_SHARED((n_pad, _D), jnp.float32),
            pltpu.VMEM((k, _CHUNK), jnp.int32),
            pltpu.VMEM((k, _CHUNK), jnp.int32),
            pltpu.VMEM((_CHUNK, _D), jnp.float32),
            pltpu.VMEM((_CHUNK, _D), jnp.float32),
            pltpu.VMEM((_CHUNK, _D), jnp.float32),
            pltpu.VMEM((_CHUNK,), jnp.float32),
            pltpu.SemaphoreType.DMA,
            pltpu.SemaphoreType.DMA,
            pltpu.SemaphoreType.DMA,
        ]
    return pl.kernel(body, out_type=out_type, mesh=mesh,
                     scratch_types=scratch,
                     compiler_params=pltpu.CompilerParams(
                         use_tc_tiling_on_sc=False))


def kernel(x, edge_index, Wl1, Wr1, b1, Wl2, Wr2, b2):
    n, d_in = x.shape
    e = edge_index.shape[1]
    n_pad = ((n + 1 + _NS * _D - 1) // (_NS * _D)) * (_NS * _D)
    k = -(-e // (_NW * _CHUNK))
    k = max(-(-k // 3) * 3, 9)
    e_pad = _NW * k * _CHUNK

    src = edge_index[0]
    dst = edge_index[1]
    dummy = n + (jnp.arange(e_pad - e, dtype=jnp.int32) % (n_pad - n))
    src_p = jnp.concatenate(
        [src, jnp.zeros((e_pad - e,), jnp.int32)]).reshape(_NW, k, _CHUNK)
    dst_p = jnp.concatenate([dst, dummy]).reshape(_NW, k, _CHUNK)

    x_p = jnp.pad(x, ((0, n_pad - n), (0, 0)))
    zrow = jnp.zeros((n_pad // _NS, _D), jnp.float32)
    zcnt = jnp.zeros((n_pad // _NS,), jnp.float32)
    ones = jnp.ones((_CHUNK,), jnp.float32)
    b1r = b1.reshape(1, _D)
    b2r = b2.reshape(1, _D)

    y1, r1 = _mm2(x_p, Wl1, Wr1)
    agg1 = _sc_agg_build(n_pad, k, True)
    part1, cnt1 = agg1(y1, src_p, dst_p, zrow, zcnt, ones)
    pa1, pb1 = part1[:n_pad], part1[n_pad:]
    ca = cnt1[:n_pad].reshape(n_pad, 1)
    cb = cnt1[n_pad:].reshape(n_pad, 1)

    y2, r2 = _combine(pa1, pb1, ca, cb, r1, Wl2, Wr2, b1r)

    agg2 = _sc_agg_build(n_pad, k, False)
    (part2,) = agg2(y2, src_p, dst_p, zrow)
    out = _final(part2[:n_pad], part2[n_pad:], ca, cb, r2, b2r)
    return out[:n]
